# baseline (speedup 1.0000x reference)
import functools

import jax
import jax.numpy as jnp
import numpy as np
from jax import lax
from jax.experimental import pallas as pl
from jax.experimental.pallas import tpu as pltpu
from jax.experimental.pallas import tpu_sc as plsc

NUM_BINS = 512
NUM_TILINGS = 8
P_BINS = int(NUM_BINS ** (2 / 3))
BATCH = 16384
LANES = 16

LO0 = np.float32(-1.2)
R0 = np.float32(np.float32(0.6) - LO0)
LO1 = np.float32(-0.07)
HI1 = np.float32(0.07)
R1 = np.float32(HI1 - LO1)
U_HI = np.float32(1.0 - 1e-6)
TABLE = NUM_BINS * NUM_BINS
TABLE_P = P_BINS ** 3

SUB = 8192
STG_T = NUM_TILINGS - 3
STG_V = STG_T * TABLE
STG_P = STG_T * TABLE_P


@functools.cache
def _build_sc_kernel():
    info = plsc.get_sparse_core_info()
    nc, ns = info.num_cores, info.num_subcores
    nw = nc * ns
    ch = BATCH // nw
    nv = ch // LANES
    g = NUM_TILINGS * ch
    g_stg = STG_T * ch
    stg_v_ch = STG_V // ns
    stg_p_ch = (-(-STG_P // ns) + 7) // 8 * 8
    spm_words = max(STG_V, ns * stg_p_ch)

    mesh = plsc.VectorSubcoreMesh(
        core_axis_name="c", subcore_axis_name="s",
        num_cores=nc, num_subcores=ns)

    f32 = jnp.float32
    out_struct = jax.ShapeDtypeStruct((BATCH,), f32)

    @functools.partial(
        pl.kernel,
        out_type=(out_struct, out_struct, out_struct),
        mesh=mesh,
        scratch_types=[
            pltpu.VMEM_SHARED((spm_words,), f32),
            pltpu.VMEM((ch,), f32),
            pltpu.VMEM((ch,), f32),
            pltpu.VMEM((ch,), f32),
            pltpu.VMEM((ch,), f32),
            pltpu.VMEM((ch,), f32),
            pltpu.VMEM((ch,), f32),
            pltpu.VMEM((ch,), f32),
            pltpu.VMEM((ch,), f32),
            pltpu.VMEM((g,), jnp.int32),
            pltpu.VMEM((g,), jnp.int32),
            pltpu.VMEM((g,), f32),
            pltpu.VMEM((g,), f32),
            pltpu.VMEM((g,), f32),
            pltpu.VMEM((SUB,), f32),
            pltpu.VMEM((SUB,), f32),
            pltpu.SemaphoreType.DMA,
            pltpu.SemaphoreType.DMA,
            pltpu.SemaphoreType.DMA,
            pltpu.SemaphoreType.DMA,
            pltpu.SemaphoreType.DMA,
            pltpu.SemaphoreType.DMA,
            pltpu.SemaphoreType.DMA,
            pltpu.SemaphoreType.DMA,
        ],
    )
    def sc_fn(p_hbm, v_hbm, wv_hbm, wr_hbm, wp_hbm,
              op_hbm, ov_hbm, or_hbm,
              spm, p_v, v_v, s0_v, s1_v, sp2_v, vp_v, pp_v, rr_v,
              idx_a, idx_b, vals_v, vals_r, vals_p, bnc0, bnc1,
              sem_si, sem_so, sem_vs, sem_vt, sem_rs, sem_rt, sem_ps, sem_pt):
        sid = lax.axis_index("s")
        wid = sid * nc + lax.axis_index("c")
        base = wid * ch
        bounce = (bnc0, bnc1)

        def stage_table(src_hbm, tile_off, n_words):
            pieces = []
            off = 0
            while off < n_words:
                pieces.append((off, min(SUB, n_words - off)))
                off += pieces[-1][1]
            outs = []
            for k, (off, sz) in enumerate(pieces):
                b = bounce[k % 2]
                if k >= 2:
                    outs[k - 2].wait()
                ci = pltpu.make_async_copy(
                    src_hbm.at[pl.ds(tile_off + off, sz)],
                    b.at[pl.ds(0, sz)], sem_si)
                ci.start()
                ci.wait()
                co = pltpu.make_async_copy(
                    b.at[pl.ds(0, sz)],
                    spm.at[pl.ds(tile_off + off, sz)], sem_so)
                co.start()
                outs.append(co)
            for co in outs[-2:]:
                co.wait()

        with jax.named_scope("ph_in"):
            pltpu.sync_copy(p_hbm.at[pl.ds(base, ch)], p_v)
            pltpu.sync_copy(v_hbm.at[pl.ds(base, ch)], v_v)

        def scale_body(i, carry):
            off = i * LANES
            p16 = p_v[pl.ds(off, LANES)]
            v16 = v_v[pl.ds(off, LANES)]
            u0 = jnp.clip((p16 - LO0) / R0, 0.0, U_HI)
            u1 = jnp.clip((v16 - LO1) / R1, 0.0, U_HI)
            s0_v[pl.ds(off, LANES)] = u0 * np.float32(NUM_BINS)
            s1_v[pl.ds(off, LANES)] = u1 * np.float32(NUM_BINS)
            return carry

        lax.fori_loop(0, nv, scale_body, 0)

        def make_idx_a_body(t):
            def idx_a_body(i, carry):
                off = i * LANES
                o = np.float32(t / NUM_TILINGS)
                s0 = s0_v[pl.ds(off, LANES)]
                s1 = s1_v[pl.ds(off, LANES)]
                i0 = jnp.minimum((s0 + o).astype(jnp.int32), NUM_BINS - 1)
                i1 = jnp.minimum((s1 + o).astype(jnp.int32), NUM_BINS - 1)
                idx_a[pl.ds(t * ch + off, LANES)] = i0 + i1 * NUM_BINS + t * TABLE
                return carry
            return idx_a_body

        for t in range(STG_T, NUM_TILINGS):
            lax.fori_loop(0, nv, make_idx_a_body(t), 0)
        tail = pl.ds(STG_T * ch, (NUM_TILINGS - STG_T) * ch)
        cp_vt = pltpu.make_async_copy(
            wv_hbm.at[idx_a.at[tail]], vals_v.at[tail], sem_vt)
        cp_vt.start()

        st_off = sid * stg_v_ch
        with jax.named_scope("ph_stage_wv"):
            stage_table(wv_hbm, st_off, stg_v_ch)

        for t in range(STG_T):
            lax.fori_loop(0, nv, make_idx_a_body(t), 0)

        plsc.subcore_barrier()

        stg = pl.ds(0, g_stg)
        cp_vs = pltpu.make_async_copy(
            spm.at[idx_a.at[stg]], vals_v.at[stg], sem_vs)
        cp_vs.start()
        cp_rt = pltpu.make_async_copy(
            wr_hbm.at[idx_a.at[tail]], vals_r.at[tail], sem_rt)
        cp_rt.start()
        with jax.named_scope("ph_wait_v"):
            cp_vs.wait()
            cp_vt.wait()

        def vprime_body(i, carry):
            off = i * LANES
            acc = vals_v[pl.ds(off, LANES)]
            for t in range(1, NUM_TILINGS):
                acc = acc + vals_v[pl.ds(t * ch + off, LANES)]
            vp = jnp.clip(v_v[pl.ds(off, LANES)] + acc, LO1, HI1)
            vp_v[pl.ds(off, LANES)] = vp
            s0_v[pl.ds(off, LANES)] = (
                s0_v[pl.ds(off, LANES)] * np.float32(P_BINS / NUM_BINS))
            s1_v[pl.ds(off, LANES)] = (
                s1_v[pl.ds(off, LANES)] * np.float32(P_BINS / NUM_BINS))
            u2 = jnp.clip((vp - LO1) / R1, 0.0, U_HI)
            sp2_v[pl.ds(off, LANES)] = u2 * np.float32(P_BINS)
            return carry

        lax.fori_loop(0, nv, vprime_body, 0)

        plsc.subcore_barrier()

        def make_idx_b_body(t):
            def idx_b_body(i, carry):
                off = i * LANES
                o = np.float32(t / NUM_TILINGS)
                i0 = jnp.minimum((s0_v[pl.ds(off, LANES)] + o).astype(jnp.int32), P_BINS - 1)
                i1 = jnp.minimum((s1_v[pl.ds(off, LANES)] + o).astype(jnp.int32), P_BINS - 1)
                i2 = jnp.minimum((sp2_v[pl.ds(off, LANES)] + o).astype(jnp.int32), P_BINS - 1)
                idx_b[pl.ds(t * ch + off, LANES)] = (
                    i0 + i1 * P_BINS + i2 * (P_BINS * P_BINS) + t * TABLE_P)
                return carry
            return idx_b_body

        for t in range(STG_T, NUM_TILINGS):
            lax.fori_loop(0, nv, make_idx_b_body(t), 0)
        cp_pt = pltpu.make_async_copy(
            wp_hbm.at[idx_b.at[tail]], vals_p.at[tail], sem_pt)
        cp_pt.start()

        for t in range(STG_T):
            lax.fori_loop(0, nv, make_idx_b_body(t), 0)

        with jax.named_scope("ph_stage_wp"):
            stage_table(wp_hbm, sid * stg_p_ch, stg_p_ch)
        plsc.subcore_barrier()

        cp_ps = pltpu.make_async_copy(
            spm.at[idx_b.at[stg]], vals_p.at[stg], sem_ps)
        cp_ps.start()
        with jax.named_scope("ph_wait_p"):
            cp_ps.wait()
            cp_pt.wait()

        def p_body(i, carry):
            off = i * LANES
            acc = vals_p[pl.ds(off, LANES)]
            for t in range(1, NUM_TILINGS):
                acc = acc + vals_p[pl.ds(t * ch + off, LANES)]
            pp_v[pl.ds(off, LANES)] = jnp.clip(
                p_v[pl.ds(off, LANES)] + acc, LO0, np.float32(0.6))
            return carry

        lax.fori_loop(0, nv, p_body, 0)

        plsc.subcore_barrier()
        with jax.named_scope("ph_stage_wr"):
            stage_table(wr_hbm, st_off, stg_v_ch)
        plsc.subcore_barrier()

        cp_rs = pltpu.make_async_copy(
            spm.at[idx_a.at[stg]], vals_r.at[stg], sem_rs)
        cp_rs.start()
        with jax.named_scope("ph_wait_r"):
            cp_rs.wait()
            cp_rt.wait()

        def r_body(i, carry):
            off = i * LANES
            acc = vals_r[pl.ds(off, LANES)]
            for t in range(1, NUM_TILINGS):
                acc = acc + vals_r[pl.ds(t * ch + off, LANES)]
            rr_v[pl.ds(off, LANES)] = acc
            return carry

        lax.fori_loop(0, nv, r_body, 0)

        pltpu.sync_copy(pp_v, op_hbm.at[pl.ds(base, ch)])
        pltpu.sync_copy(vp_v, ov_hbm.at[pl.ds(base, ch)])
        pltpu.sync_copy(rr_v, or_hbm.at[pl.ds(base, ch)])

    return sc_fn


def kernel(state, action, Wp, Wv, Wr):
    del action
    sc_fn = _build_sc_kernel()
    p = state[:, 0]
    v = state[:, 1]
    pp, vp, rr = sc_fn(p, v, Wv.reshape(-1), Wr.reshape(-1), Wp.reshape(-1))
    return jnp.stack([pp, vp, rr], axis=1)


# speedup vs baseline: 1.4530x; 1.4530x over previous
import functools

import jax
import jax.numpy as jnp
import numpy as np
from jax import lax
from jax.experimental import pallas as pl
from jax.experimental.pallas import tpu as pltpu
from jax.experimental.pallas import tpu_sc as plsc

NUM_BINS = 512
NUM_TILINGS = 8
P_BINS = int(NUM_BINS ** (2 / 3))
BATCH = 16384
LANES = 16

LO0 = np.float32(-1.2)
R0 = np.float32(np.float32(0.6) - LO0)
LO1 = np.float32(-0.07)
HI1 = np.float32(0.07)
R1 = np.float32(HI1 - LO1)
U_HI = np.float32(1.0 - 1e-6)
TABLE = NUM_BINS * NUM_BINS
TABLE_P = P_BINS ** 3

SUB = 8192
NS_T = 16

ROT_T = 3
REG1_WV7 = 0
REG1_WR7 = TABLE
REG2 = 2 * TABLE
SPM_WORDS = REG2 + ROT_T * TABLE

V_ROUNDS = [(0, 3), (3, 3), (6, 1)]
P_ROUNDS = [(0, 3), (3, 3), (6, 2)]

WP_CHUNKS = [(-(-(c * TABLE_P) // NS_T) + 7) // 8 * 8 for _, c in P_ROUNDS]
WP_OFFS = [0]
for _c in WP_CHUNKS[:-1]:
    WP_OFFS.append(WP_OFFS[-1] + NS_T * _c)


@functools.cache
def _build_sc_kernel():
    info = plsc.get_sparse_core_info()
    nc, ns = info.num_cores, info.num_subcores
    nw = nc * ns
    assert ns == NS_T
    ch = BATCH // nw
    nv = ch // LANES
    g = NUM_TILINGS * ch

    mesh = plsc.VectorSubcoreMesh(
        core_axis_name="c", subcore_axis_name="s",
        num_cores=nc, num_subcores=ns)

    f32 = jnp.float32
    out_struct = jax.ShapeDtypeStruct((BATCH,), f32)

    @functools.partial(
        pl.kernel,
        out_type=(out_struct, out_struct, out_struct),
        mesh=mesh,
        scratch_types=[
            pltpu.VMEM_SHARED((SPM_WORDS,), f32),
            pltpu.VMEM((ch,), f32),
            pltpu.VMEM((ch,), f32),
            pltpu.VMEM((ch,), f32),
            pltpu.VMEM((ch,), f32),
            pltpu.VMEM((ch,), f32),
            pltpu.VMEM((ch,), f32),
            pltpu.VMEM((ch,), f32),
            pltpu.VMEM((ch,), f32),
            pltpu.VMEM((g,), jnp.int32),
            pltpu.VMEM((ch,), jnp.int32),
            pltpu.VMEM((g,), jnp.int32),
            pltpu.VMEM((g,), f32),
            pltpu.VMEM((g,), f32),
            pltpu.VMEM((g,), f32),
            pltpu.VMEM((SUB,), f32),
            pltpu.VMEM((SUB,), f32),
            pltpu.SemaphoreType.DMA,
            pltpu.SemaphoreType.DMA,
            pltpu.SemaphoreType.DMA,
            pltpu.SemaphoreType.DMA,
            pltpu.SemaphoreType.DMA,
            pltpu.SemaphoreType.DMA,
            pltpu.SemaphoreType.DMA,
        ],
    )
    def sc_fn(p_hbm, v_hbm, wv_hbm, wr_hbm, wp_hbm,
              op_hbm, ov_hbm, or_hbm,
              spm, p_v, v_v, s0_v, s1_v, sp2_v, vp_v, pp_v, rr_v,
              idx_a, idx_rt, idx_b, vals_v, vals_r, vals_p, bnc0, bnc1,
              sem_si, sem_so, sem_v, sem_r, sem_p, sem_vt, sem_rt):
        sid = lax.axis_index("s")
        wid = sid * nc + lax.axis_index("c")
        base = wid * ch
        bounce = (bnc0, bnc1)

        def stage(src_hbm, src_off, dst_off, n_words):
            pieces = []
            off = 0
            while off < n_words:
                pieces.append((off, min(SUB, n_words - off)))
                off += pieces[-1][1]
            outs = []
            for k, (off, sz) in enumerate(pieces):
                b = bounce[k % 2]
                if k >= 2:
                    outs[k - 2].wait()
                ci = pltpu.make_async_copy(
                    src_hbm.at[pl.ds(src_off + sid * n_words + off, sz)],
                    b.at[pl.ds(0, sz)], sem_si)
                ci.start()
                ci.wait()
                co = pltpu.make_async_copy(
                    b.at[pl.ds(0, sz)],
                    spm.at[pl.ds(dst_off + sid * n_words + off, sz)], sem_so)
                co.start()
                outs.append(co)
            for co in outs[-2:]:
                co.wait()

        def gather(idx_ref, lo, n, vals_ref, sem):
            sl = pl.ds(lo, n)
            cp = pltpu.make_async_copy(
                spm.at[idx_ref.at[sl]], vals_ref.at[sl], sem)
            cp.start()
            return cp

        with jax.named_scope("ph_in"):
            pltpu.sync_copy(p_hbm.at[pl.ds(base, ch)], p_v)
            pltpu.sync_copy(v_hbm.at[pl.ds(base, ch)], v_v)

        def scale_body(i, carry):
            off = i * LANES
            p16 = p_v[pl.ds(off, LANES)]
            v16 = v_v[pl.ds(off, LANES)]
            u0 = jnp.clip((p16 - LO0) / R0, 0.0, U_HI)
            u1 = jnp.clip((v16 - LO1) / R1, 0.0, U_HI)
            s0_v[pl.ds(off, LANES)] = u0 * np.float32(NUM_BINS)
            s1_v[pl.ds(off, LANES)] = u1 * np.float32(NUM_BINS)
            return carry

        lax.fori_loop(0, nv, scale_body, 0)

        def vr_rel(t):
            if t == NUM_TILINGS - 1:
                return REG1_WV7
            for s0r, cnt in V_ROUNDS:
                if s0r <= t < s0r + cnt:
                    return REG2 + (t - s0r) * TABLE

        def make_idx_a_body(t):
            rel = vr_rel(t)

            def idx_a_body(i, carry):
                off = i * LANES
                o = np.float32(t / NUM_TILINGS)
                s0 = s0_v[pl.ds(off, LANES)]
                s1 = s1_v[pl.ds(off, LANES)]
                i0 = jnp.minimum((s0 + o).astype(jnp.int32), NUM_BINS - 1)
                i1 = jnp.minimum((s1 + o).astype(jnp.int32), NUM_BINS - 1)
                flat = i0 + i1 * NUM_BINS
                idx_a[pl.ds(t * ch + off, LANES)] = flat + rel
                if t == NUM_TILINGS - 1:
                    idx_rt[pl.ds(off, LANES)] = flat + REG1_WR7
                return carry
            return idx_a_body

        for t in range(NUM_TILINGS):
            lax.fori_loop(0, nv, make_idx_a_body(t), 0)

        with jax.named_scope("ph_stage_tails"):
            stage(wv_hbm, (NUM_TILINGS - 1) * TABLE, REG1_WV7, TABLE // ns)
            stage(wr_hbm, (NUM_TILINGS - 1) * TABLE, REG1_WR7, TABLE // ns)
        plsc.subcore_barrier()
        cp_vt = gather(idx_a, (NUM_TILINGS - 1) * ch, ch, vals_v, sem_vt)
        cp_rt = gather(
            idx_rt, 0, ch,
            vals_r.at[pl.ds((NUM_TILINGS - 1) * ch, ch)], sem_rt)

        for s0r, cnt in V_ROUNDS:
            with jax.named_scope("ph_stage_wv"):
                stage(wv_hbm, s0r * TABLE, REG2, cnt * TABLE // ns)
            plsc.subcore_barrier()
            cp = gather(idx_a, s0r * ch, cnt * ch, vals_v, sem_v)
            with jax.named_scope("ph_wait_v"):
                cp.wait()
            plsc.subcore_barrier()
        with jax.named_scope("ph_wait_vt"):
            cp_vt.wait()

        def vprime_body(i, carry):
            off = i * LANES
            acc = vals_v[pl.ds(off, LANES)]
            for t in range(1, NUM_TILINGS):
                acc = acc + vals_v[pl.ds(t * ch + off, LANES)]
            vp = jnp.clip(v_v[pl.ds(off, LANES)] + acc, LO1, HI1)
            vp_v[pl.ds(off, LANES)] = vp
            s0_v[pl.ds(off, LANES)] = (
                s0_v[pl.ds(off, LANES)] * np.float32(P_BINS / NUM_BINS))
            s1_v[pl.ds(off, LANES)] = (
                s1_v[pl.ds(off, LANES)] * np.float32(P_BINS / NUM_BINS))
            u2 = jnp.clip((vp - LO1) / R1, 0.0, U_HI)
            sp2_v[pl.ds(off, LANES)] = u2 * np.float32(P_BINS)
            return carry

        lax.fori_loop(0, nv, vprime_body, 0)

        def p_rel(t):
            for s0r, cnt in P_ROUNDS:
                if s0r <= t < s0r + cnt:
                    return REG2 + (t - s0r) * TABLE_P

        def make_idx_b_body(t):
            rel = p_rel(t)

            def idx_b_body(i, carry):
                off = i * LANES
                o = np.float32(t / NUM_TILINGS)
                i0 = jnp.minimum((s0_v[pl.ds(off, LANES)] + o).astype(jnp.int32), P_BINS - 1)
                i1 = jnp.minimum((s1_v[pl.ds(off, LANES)] + o).astype(jnp.int32), P_BINS - 1)
                i2 = jnp.minimum((sp2_v[pl.ds(off, LANES)] + o).astype(jnp.int32), P_BINS - 1)
                idx_b[pl.ds(t * ch + off, LANES)] = (
                    i0 + i1 * P_BINS + i2 * (P_BINS * P_BINS) + rel)
                return carry
            return idx_b_body

        for t in range(NUM_TILINGS):
            lax.fori_loop(0, nv, make_idx_b_body(t), 0)

        plsc.subcore_barrier()
        for (s0r, cnt), seg, chunk in zip(P_ROUNDS, WP_OFFS, WP_CHUNKS):
            with jax.named_scope("ph_stage_wp"):
                stage(wp_hbm, seg, REG2, chunk)
            plsc.subcore_barrier()
            cp = gather(idx_b, s0r * ch, cnt * ch, vals_p, sem_p)
            with jax.named_scope("ph_wait_p"):
                cp.wait()
            plsc.subcore_barrier()

        def p_body(i, carry):
            off = i * LANES
            acc = vals_p[pl.ds(off, LANES)]
            for t in range(1, NUM_TILINGS):
                acc = acc + vals_p[pl.ds(t * ch + off, LANES)]
            pp_v[pl.ds(off, LANES)] = jnp.clip(
                p_v[pl.ds(off, LANES)] + acc, LO0, np.float32(0.6))
            return carry

        lax.fori_loop(0, nv, p_body, 0)

        for s0r, cnt in V_ROUNDS:
            with jax.named_scope("ph_stage_wr"):
                stage(wr_hbm, s0r * TABLE, REG2, cnt * TABLE // ns)
            plsc.subcore_barrier()
            cp = gather(idx_a, s0r * ch, cnt * ch, vals_r, sem_r)
            with jax.named_scope("ph_wait_r"):
                cp.wait()
            plsc.subcore_barrier()
        with jax.named_scope("ph_wait_rt"):
            cp_rt.wait()

        def r_body(i, carry):
            off = i * LANES
            acc = vals_r[pl.ds(off, LANES)]
            for t in range(1, NUM_TILINGS):
                acc = acc + vals_r[pl.ds(t * ch + off, LANES)]
            rr_v[pl.ds(off, LANES)] = acc
            return carry

        lax.fori_loop(0, nv, r_body, 0)

        pltpu.sync_copy(pp_v, op_hbm.at[pl.ds(base, ch)])
        pltpu.sync_copy(vp_v, ov_hbm.at[pl.ds(base, ch)])
        pltpu.sync_copy(rr_v, or_hbm.at[pl.ds(base, ch)])

    return sc_fn


def kernel(state, action, Wp, Wv, Wr):
    del action
    sc_fn = _build_sc_kernel()
    p = state[:, 0]
    v = state[:, 1]
    segs = []
    for (s0r, cnt), chunk in zip(P_ROUNDS, WP_CHUNKS):
        seg = Wp[s0r:s0r + cnt].reshape(-1)
        segs.append(jnp.pad(seg, (0, NS_T * chunk - cnt * TABLE_P)))
    pp, vp, rr = sc_fn(p, v, Wv.reshape(-1), Wr.reshape(-1),
                       jnp.concatenate(segs))
    return jnp.stack([pp, vp, rr], axis=1)


# speedup vs baseline: 1.6903x; 1.1634x over previous
import functools

import jax
import jax.numpy as jnp
import numpy as np
from jax import lax
from jax.experimental import pallas as pl
from jax.experimental.pallas import tpu as pltpu
from jax.experimental.pallas import tpu_sc as plsc

NUM_BINS = 512
NUM_TILINGS = 8
P_BINS = int(NUM_BINS ** (2 / 3))
BATCH = 16384
LANES = 16

LO0 = np.float32(-1.2)
R0 = np.float32(np.float32(0.6) - LO0)
LO1 = np.float32(-0.07)
HI1 = np.float32(0.07)
R1 = np.float32(HI1 - LO1)
U_HI = np.float32(1.0 - 1e-6)
TABLE = NUM_BINS * NUM_BINS
TABLE_P = P_BINS ** 3

SUB = 8192
NS_T = 16

ROT_T = 3
REG1_WV7 = 0
REG1_WR7 = TABLE
REG2 = 2 * TABLE
SPM_WORDS = REG2 + ROT_T * TABLE

V_ROUNDS = [(0, 3), (3, 3), (6, 1)]
P_ROUNDS = [(0, 3), (3, 3), (6, 2)]

P_SEGS = []
for _s0, _cnt in P_ROUNDS:
    _start = _s0 * TABLE_P // 8 * 8
    _shift = _s0 * TABLE_P - _start
    _chunk = (-(-(_cnt * TABLE_P + _shift) // NS_T) + 7) // 8 * 8
    assert _start + NS_T * _chunk <= NUM_TILINGS * TABLE_P + 63
    P_SEGS.append((_start, _shift, _chunk))


@functools.cache
def _build_sc_kernel():
    info = plsc.get_sparse_core_info()
    nc, ns = info.num_cores, info.num_subcores
    nw = nc * ns
    assert ns == NS_T
    ch = BATCH // nw
    nv = ch // LANES
    g = NUM_TILINGS * ch

    mesh = plsc.VectorSubcoreMesh(
        core_axis_name="c", subcore_axis_name="s",
        num_cores=nc, num_subcores=ns)

    f32 = jnp.float32
    out_struct = jax.ShapeDtypeStruct((BATCH,), f32)

    @functools.partial(
        pl.kernel,
        out_type=(out_struct, out_struct, out_struct),
        mesh=mesh,
        scratch_types=[
            pltpu.VMEM_SHARED((SPM_WORDS,), f32),
            pltpu.VMEM((ch,), f32),
            pltpu.VMEM((ch,), f32),
            pltpu.VMEM((ch,), f32),
            pltpu.VMEM((ch,), f32),
            pltpu.VMEM((ch,), f32),
            pltpu.VMEM((ch,), f32),
            pltpu.VMEM((ch,), f32),
            pltpu.VMEM((ch,), f32),
            pltpu.VMEM((g,), jnp.int32),
            pltpu.VMEM((ch,), jnp.int32),
            pltpu.VMEM((g,), jnp.int32),
            pltpu.VMEM((g,), f32),
            pltpu.VMEM((g,), f32),
            pltpu.VMEM((g,), f32),
            pltpu.VMEM((SUB,), f32),
            pltpu.VMEM((SUB,), f32),
            pltpu.SemaphoreType.DMA,
            pltpu.SemaphoreType.DMA,
            pltpu.SemaphoreType.DMA,
            pltpu.SemaphoreType.DMA,
            pltpu.SemaphoreType.DMA,
            pltpu.SemaphoreType.DMA,
            pltpu.SemaphoreType.DMA,
        ],
    )
    def sc_fn(p_hbm, v_hbm, wv_hbm, wr_hbm, wp_hbm,
              op_hbm, ov_hbm, or_hbm,
              spm, p_v, v_v, s0_v, s1_v, sp2_v, vp_v, pp_v, rr_v,
              idx_a, idx_rt, idx_b, vals_v, vals_r, vals_p, bnc0, bnc1,
              sem_si, sem_so, sem_v, sem_r, sem_p, sem_vt, sem_rt):
        sid = lax.axis_index("s")
        wid = sid * nc + lax.axis_index("c")
        base = wid * ch
        bounce = (bnc0, bnc1)

        def stage(src_hbm, src_off, dst_off, n_words):
            pieces = []
            off = 0
            while off < n_words:
                pieces.append((off, min(SUB, n_words - off)))
                off += pieces[-1][1]
            outs = []
            for k, (off, sz) in enumerate(pieces):
                b = bounce[k % 2]
                if k >= 2:
                    outs[k - 2].wait()
                ci = pltpu.make_async_copy(
                    src_hbm.at[pl.ds(src_off + sid * n_words + off, sz)],
                    b.at[pl.ds(0, sz)], sem_si)
                ci.start()
                ci.wait()
                co = pltpu.make_async_copy(
                    b.at[pl.ds(0, sz)],
                    spm.at[pl.ds(dst_off + sid * n_words + off, sz)], sem_so)
                co.start()
                outs.append(co)
            for co in outs[-2:]:
                co.wait()

        def gather(idx_ref, lo, n, vals_ref, sem):
            sl = pl.ds(lo, n)
            cp = pltpu.make_async_copy(
                spm.at[idx_ref.at[sl]], vals_ref.at[sl], sem)
            cp.start()
            return cp

        with jax.named_scope("ph_in"):
            pltpu.sync_copy(p_hbm.at[pl.ds(base, ch)], p_v)
            pltpu.sync_copy(v_hbm.at[pl.ds(base, ch)], v_v)

        def scale_body(i, carry):
            off = i * LANES
            p16 = p_v[pl.ds(off, LANES)]
            v16 = v_v[pl.ds(off, LANES)]
            u0 = jnp.clip((p16 - LO0) / R0, 0.0, U_HI)
            u1 = jnp.clip((v16 - LO1) / R1, 0.0, U_HI)
            s0_v[pl.ds(off, LANES)] = u0 * np.float32(NUM_BINS)
            s1_v[pl.ds(off, LANES)] = u1 * np.float32(NUM_BINS)
            return carry

        lax.fori_loop(0, nv, scale_body, 0)

        def vr_rel(t):
            if t == NUM_TILINGS - 1:
                return REG1_WV7
            for s0r, cnt in V_ROUNDS:
                if s0r <= t < s0r + cnt:
                    return REG2 + (t - s0r) * TABLE

        def make_idx_a_body(t):
            rel = vr_rel(t)

            def idx_a_body(i, carry):
                off = i * LANES
                o = np.float32(t / NUM_TILINGS)
                s0 = s0_v[pl.ds(off, LANES)]
                s1 = s1_v[pl.ds(off, LANES)]
                i0 = jnp.minimum((s0 + o).astype(jnp.int32), NUM_BINS - 1)
                i1 = jnp.minimum((s1 + o).astype(jnp.int32), NUM_BINS - 1)
                flat = i0 + i1 * NUM_BINS
                idx_a[pl.ds(t * ch + off, LANES)] = flat + rel
                if t == NUM_TILINGS - 1:
                    idx_rt[pl.ds(off, LANES)] = flat + REG1_WR7
                return carry
            return idx_a_body

        for t in range(NUM_TILINGS):
            lax.fori_loop(0, nv, make_idx_a_body(t), 0)

        with jax.named_scope("ph_stage_tails"):
            stage(wv_hbm, (NUM_TILINGS - 1) * TABLE, REG1_WV7, TABLE // ns)
            stage(wr_hbm, (NUM_TILINGS - 1) * TABLE, REG1_WR7, TABLE // ns)
        plsc.subcore_barrier()
        cp_vt = gather(idx_a, (NUM_TILINGS - 1) * ch, ch, vals_v, sem_vt)
        cp_rt = gather(
            idx_rt, 0, ch,
            vals_r.at[pl.ds((NUM_TILINGS - 1) * ch, ch)], sem_rt)

        for s0r, cnt in V_ROUNDS:
            with jax.named_scope("ph_stage_wv"):
                stage(wv_hbm, s0r * TABLE, REG2, cnt * TABLE // ns)
            plsc.subcore_barrier()
            cp = gather(idx_a, s0r * ch, cnt * ch, vals_v, sem_v)
            with jax.named_scope("ph_wait_v"):
                cp.wait()
            plsc.subcore_barrier()
        with jax.named_scope("ph_wait_vt"):
            cp_vt.wait()

        def vprime_body(i, carry):
            off = i * LANES
            acc = vals_v[pl.ds(off, LANES)]
            for t in range(1, NUM_TILINGS):
                acc = acc + vals_v[pl.ds(t * ch + off, LANES)]
            vp = jnp.clip(v_v[pl.ds(off, LANES)] + acc, LO1, HI1)
            vp_v[pl.ds(off, LANES)] = vp
            s0_v[pl.ds(off, LANES)] = (
                s0_v[pl.ds(off, LANES)] * np.float32(P_BINS / NUM_BINS))
            s1_v[pl.ds(off, LANES)] = (
                s1_v[pl.ds(off, LANES)] * np.float32(P_BINS / NUM_BINS))
            u2 = jnp.clip((vp - LO1) / R1, 0.0, U_HI)
            sp2_v[pl.ds(off, LANES)] = u2 * np.float32(P_BINS)
            return carry

        lax.fori_loop(0, nv, vprime_body, 0)

        def p_rel(t):
            for (s0r, cnt), (_, shift, _c) in zip(P_ROUNDS, P_SEGS):
                if s0r <= t < s0r + cnt:
                    return REG2 + shift + (t - s0r) * TABLE_P

        def make_idx_b_body(t):
            rel = p_rel(t)

            def idx_b_body(i, carry):
                off = i * LANES
                o = np.float32(t / NUM_TILINGS)
                i0 = jnp.minimum((s0_v[pl.ds(off, LANES)] + o).astype(jnp.int32), P_BINS - 1)
                i1 = jnp.minimum((s1_v[pl.ds(off, LANES)] + o).astype(jnp.int32), P_BINS - 1)
                i2 = jnp.minimum((sp2_v[pl.ds(off, LANES)] + o).astype(jnp.int32), P_BINS - 1)
                idx_b[pl.ds(t * ch + off, LANES)] = (
                    i0 + i1 * P_BINS + i2 * (P_BINS * P_BINS) + rel)
                return carry
            return idx_b_body

        for t in range(NUM_TILINGS):
            lax.fori_loop(0, nv, make_idx_b_body(t), 0)

        plsc.subcore_barrier()
        for (s0r, cnt), (seg_start, _, seg_chunk) in zip(P_ROUNDS, P_SEGS):
            with jax.named_scope("ph_stage_wp"):
                stage(wp_hbm, seg_start, REG2, seg_chunk)
            plsc.subcore_barrier()
            cp = gather(idx_b, s0r * ch, cnt * ch, vals_p, sem_p)
            with jax.named_scope("ph_wait_p"):
                cp.wait()
            plsc.subcore_barrier()

        def p_body(i, carry):
            off = i * LANES
            acc = vals_p[pl.ds(off, LANES)]
            for t in range(1, NUM_TILINGS):
                acc = acc + vals_p[pl.ds(t * ch + off, LANES)]
            pp_v[pl.ds(off, LANES)] = jnp.clip(
                p_v[pl.ds(off, LANES)] + acc, LO0, np.float32(0.6))
            return carry

        lax.fori_loop(0, nv, p_body, 0)

        for s0r, cnt in V_ROUNDS:
            with jax.named_scope("ph_stage_wr"):
                stage(wr_hbm, s0r * TABLE, REG2, cnt * TABLE // ns)
            plsc.subcore_barrier()
            cp = gather(idx_a, s0r * ch, cnt * ch, vals_r, sem_r)
            with jax.named_scope("ph_wait_r"):
                cp.wait()
            plsc.subcore_barrier()
        with jax.named_scope("ph_wait_rt"):
            cp_rt.wait()

        def r_body(i, carry):
            off = i * LANES
            acc = vals_r[pl.ds(off, LANES)]
            for t in range(1, NUM_TILINGS):
                acc = acc + vals_r[pl.ds(t * ch + off, LANES)]
            rr_v[pl.ds(off, LANES)] = acc
            return carry

        lax.fori_loop(0, nv, r_body, 0)

        pltpu.sync_copy(pp_v, op_hbm.at[pl.ds(base, ch)])
        pltpu.sync_copy(vp_v, ov_hbm.at[pl.ds(base, ch)])
        pltpu.sync_copy(rr_v, or_hbm.at[pl.ds(base, ch)])

    return sc_fn


def kernel(state, action, Wp, Wv, Wr):
    del action
    sc_fn = _build_sc_kernel()
    p = state[:, 0]
    v = state[:, 1]
    pp, vp, rr = sc_fn(p, v, Wv.reshape(-1), Wr.reshape(-1), Wp.reshape(-1))
    return jnp.stack([pp, vp, rr], axis=1)
